# Initial kernel scaffold; baseline (speedup 1.0000x reference)
import jax
import jax.numpy as jnp
from jax.experimental import pallas as pl


def kernel(path, shared, ctx_g, ctx_c, W_shared_w, W_shared_b, w_gate, token_prefix, token_suffix, tokenized_prompts):
    raise NotImplementedError("write your pallas kernel here")


# speedup vs baseline: 2.2110x; 2.2110x over previous
import jax
import jax.numpy as jnp
from jax import lax
from jax.experimental import pallas as pl
from jax.experimental.pallas import tpu as pltpu

N_CLS = 50
N_CTX = 16
CTX_DIM = 768
N_EXPERTS = 64
TOP_K = 4
CONTEXT_LEN = 128
HALF = N_CTX // 2
NC_ROWS = HALF - 1
SUF = CONTEXT_LEN - 1 - N_CTX


def _fused_body(path_ref, shared_ref, ctx_g_ref, ctx_c_ref, w_ref, b_ref,
                wg_ref, pre_ref, suf_ref, out_ref, aux_ref, mid_ref):
    c = pl.program_id(0)

    @pl.when(c == 0)
    def _compute():
        ctx_s = lax.dot_general(
            shared_ref[...], w_ref[...], (((1,), (1,)), ((), ())),
            preferred_element_type=jnp.float32) + b_ref[...]

        logits = lax.dot_general(
            path_ref[...], wg_ref[...], (((1,), (0,)), ((), ())),
            preferred_element_type=jnp.float32)

        iota = lax.broadcasted_iota(jnp.int32, (1, N_EXPERTS), 1)
        work = logits
        top_mask = jnp.zeros((1, N_EXPERTS), jnp.bool_)
        vmax = jnp.max(work)
        for _ in range(TOP_K):
            m = jnp.max(work)
            sel = jnp.min(jnp.where(work == m, iota, N_EXPERTS))
            mk = iota == sel
            top_mask = jnp.logical_or(top_mask, mk)
            work = jnp.where(mk, -jnp.inf, work)

        e = jnp.where(top_mask, jnp.exp(logits - vmax), 0.0)
        gates = e / jnp.sum(e)

        eps = 1e-10
        imp_mean = jnp.sum(gates) / N_EXPERTS
        imp_var = jnp.sum((gates - imp_mean) ** 2) / N_EXPERTS
        load = (gates > 0).astype(jnp.float32)
        load_mean = jnp.sum(load) / N_EXPERTS
        load_var = jnp.sum((load - load_mean) ** 2) / N_EXPERTS
        aux = imp_var / (imp_mean ** 2 + eps) + load_var / (load_mean ** 2 + eps)
        aux_ref[...] = jnp.full((1, 1), aux, jnp.float32)

        mid_ref[0:HALF, :] = ctx_g_ref[...]
        for j in range(NC_ROWS):
            mid_ref[HALF + j:HALF + j + 1, :] = lax.dot_general(
                gates, ctx_c_ref[:, j, :], (((1,), (0,)), ((), ())),
                preferred_element_type=jnp.float32)
        mid_ref[N_CTX - 1:N_CTX, :] = ctx_s

    out_ref[0, 0:1, :] = pre_ref[0]
    out_ref[0, 1:1 + N_CTX, :] = mid_ref[...]
    out_ref[0, 1 + N_CTX:, :] = suf_ref[0]


def kernel(path, shared, ctx_g, ctx_c, W_shared_w, W_shared_b, w_gate,
           token_prefix, token_suffix, tokenized_prompts):
    ctx_c3 = ctx_c.reshape(N_EXPERTS, NC_ROWS, CTX_DIM)
    b2 = W_shared_b.reshape(1, CTX_DIM)
    prompts, aux = pl.pallas_call(
        _fused_body,
        grid=(N_CLS,),
        in_specs=[
            pl.BlockSpec((1, 512), lambda c: (0, 0)),
            pl.BlockSpec((1, 256), lambda c: (0, 0)),
            pl.BlockSpec((HALF, CTX_DIM), lambda c: (0, 0)),
            pl.BlockSpec((N_EXPERTS, NC_ROWS, CTX_DIM), lambda c: (0, 0, 0)),
            pl.BlockSpec((CTX_DIM, 256), lambda c: (0, 0)),
            pl.BlockSpec((1, CTX_DIM), lambda c: (0, 0)),
            pl.BlockSpec((512, N_EXPERTS), lambda c: (0, 0)),
            pl.BlockSpec((1, 1, CTX_DIM), lambda c: (c, 0, 0)),
            pl.BlockSpec((1, SUF, CTX_DIM), lambda c: (c, 0, 0)),
        ],
        out_specs=[
            pl.BlockSpec((1, CONTEXT_LEN, CTX_DIM), lambda c: (c, 0, 0)),
            pl.BlockSpec((1, 1), lambda c: (0, 0)),
        ],
        out_shape=[
            jax.ShapeDtypeStruct((N_CLS, CONTEXT_LEN, CTX_DIM), jnp.float32),
            jax.ShapeDtypeStruct((1, 1), jnp.float32),
        ],
        scratch_shapes=[pltpu.VMEM((N_CTX, CTX_DIM), jnp.float32)],
    )(path, shared, ctx_g, ctx_c3, W_shared_w, b2, w_gate,
      token_prefix, token_suffix)
    return prompts, tokenized_prompts, aux.reshape(())


# speedup vs baseline: 2.2131x; 1.0009x over previous
import jax
import jax.numpy as jnp
from jax import lax
from jax.experimental import pallas as pl
from jax.experimental.pallas import tpu as pltpu

N_CLS = 50
N_CTX = 16
CTX_DIM = 768
N_EXPERTS = 64
TOP_K = 4
CONTEXT_LEN = 128
HALF = N_CTX // 2
NC_ROWS = HALF - 1
SUF = CONTEXT_LEN - 1 - N_CTX


def _fused_body(path_ref, shared_ref, ctx_g_ref, ctx_c_ref, w_ref, b_ref,
                wg_ref, pre_ref, suf_ref, out_ref, aux_ref, mid_ref):
    c = pl.program_id(0)

    @pl.when(c == 0)
    def _compute():
        ctx_s = lax.dot_general(
            shared_ref[...], w_ref[...], (((1,), (1,)), ((), ())),
            preferred_element_type=jnp.float32) + b_ref[...]

        logits = lax.dot_general(
            path_ref[...], wg_ref[...], (((1,), (0,)), ((), ())),
            preferred_element_type=jnp.float32)

        iota = lax.broadcasted_iota(jnp.int32, (1, N_EXPERTS), 1)
        work = logits
        top_mask = jnp.zeros((1, N_EXPERTS), jnp.bool_)
        vmax = jnp.max(work)
        for _ in range(TOP_K):
            m = jnp.max(work)
            sel = jnp.min(jnp.where(work == m, iota, N_EXPERTS))
            mk = iota == sel
            top_mask = jnp.logical_or(top_mask, mk)
            work = jnp.where(mk, -jnp.inf, work)

        e = jnp.where(top_mask, jnp.exp(logits - vmax), 0.0)
        gates = e / jnp.sum(e)

        eps = 1e-10
        imp_mean = jnp.sum(gates) / N_EXPERTS
        imp_var = jnp.sum((gates - imp_mean) ** 2) / N_EXPERTS
        load = (gates > 0).astype(jnp.float32)
        load_mean = jnp.sum(load) / N_EXPERTS
        load_var = jnp.sum((load - load_mean) ** 2) / N_EXPERTS
        aux = imp_var / (imp_mean ** 2 + eps) + load_var / (load_mean ** 2 + eps)
        aux_ref[...] = jnp.full((1, 1), aux, jnp.float32)

        mid_ref[1:1 + HALF, :] = ctx_g_ref[...]
        for j in range(NC_ROWS):
            mid_ref[1 + HALF + j:2 + HALF + j, :] = lax.dot_general(
                gates, ctx_c_ref[:, j, :], (((1,), (0,)), ((), ())),
                preferred_element_type=jnp.float32)
        mid_ref[N_CTX:N_CTX + 1, :] = ctx_s

    head = mid_ref[0:N_CTX, :]
    rowid = lax.broadcasted_iota(jnp.int32, (N_CTX, CTX_DIM), 0)
    prow = jnp.broadcast_to(pre_ref[0], (N_CTX, CTX_DIM))
    out_ref[0, 0:N_CTX, :] = jnp.where(rowid == 0, prow, head)

    tail = jnp.concatenate([suf_ref[0], mid_ref[N_CTX:N_CTX + 1, :]], axis=0)
    out_ref[0, N_CTX:, :] = pltpu.roll(tail, 1, 0)


def kernel(path, shared, ctx_g, ctx_c, W_shared_w, W_shared_b, w_gate,
           token_prefix, token_suffix, tokenized_prompts):
    ctx_c3 = ctx_c.reshape(N_EXPERTS, NC_ROWS, CTX_DIM)
    b2 = W_shared_b.reshape(1, CTX_DIM)
    prompts, aux = pl.pallas_call(
        _fused_body,
        grid=(N_CLS,),
        in_specs=[
            pl.BlockSpec((1, 512), lambda c: (0, 0)),
            pl.BlockSpec((1, 256), lambda c: (0, 0)),
            pl.BlockSpec((HALF, CTX_DIM), lambda c: (0, 0)),
            pl.BlockSpec((N_EXPERTS, NC_ROWS, CTX_DIM), lambda c: (0, 0, 0)),
            pl.BlockSpec((CTX_DIM, 256), lambda c: (0, 0)),
            pl.BlockSpec((1, CTX_DIM), lambda c: (0, 0)),
            pl.BlockSpec((512, N_EXPERTS), lambda c: (0, 0)),
            pl.BlockSpec((1, 1, CTX_DIM), lambda c: (c, 0, 0)),
            pl.BlockSpec((1, SUF, CTX_DIM), lambda c: (c, 0, 0)),
        ],
        out_specs=[
            pl.BlockSpec((1, CONTEXT_LEN, CTX_DIM), lambda c: (c, 0, 0)),
            pl.BlockSpec((1, 1), lambda c: (0, 0)),
        ],
        out_shape=[
            jax.ShapeDtypeStruct((N_CLS, CONTEXT_LEN, CTX_DIM), jnp.float32),
            jax.ShapeDtypeStruct((1, 1), jnp.float32),
        ],
        scratch_shapes=[pltpu.VMEM((N_CTX + 8, CTX_DIM), jnp.float32)],
    )(path, shared, ctx_g, ctx_c3, W_shared_w, b2, w_gate,
      token_prefix, token_suffix)
    return prompts, tokenized_prompts, aux.reshape(())


# speedup vs baseline: 4.2102x; 1.9024x over previous
import jax
import jax.numpy as jnp
from jax import lax
from jax.experimental import pallas as pl
from jax.experimental.pallas import tpu as pltpu

N_CLS = 50
N_CTX = 16
CTX_DIM = 768
N_EXPERTS = 64
TOP_K = 4
CONTEXT_LEN = 128
HALF = N_CTX // 2
NC_ROWS = HALF - 1
SUF = CONTEXT_LEN - 1 - N_CTX
BLK = 10


def _fused_body(path_ref, shared_ref, ctx_g_ref, ctx_c_ref, w_ref, b_ref,
                wg_ref, pre_ref, suf_ref, out_ref, aux_ref, mid_ref):
    c = pl.program_id(0)

    @pl.when(c == 0)
    def _compute():
        ctx_s = lax.dot_general(
            shared_ref[...], w_ref[...], (((1,), (1,)), ((), ())),
            preferred_element_type=jnp.float32) + b_ref[...]

        logits = lax.dot_general(
            path_ref[...], wg_ref[...], (((1,), (0,)), ((), ())),
            preferred_element_type=jnp.float32)

        iota = lax.broadcasted_iota(jnp.int32, (1, N_EXPERTS), 1)
        work = logits
        top_mask = jnp.zeros((1, N_EXPERTS), jnp.bool_)
        vmax = jnp.max(work)
        for _ in range(TOP_K):
            m = jnp.max(work)
            sel = jnp.min(jnp.where(work == m, iota, N_EXPERTS))
            mk = iota == sel
            top_mask = jnp.logical_or(top_mask, mk)
            work = jnp.where(mk, -jnp.inf, work)

        e = jnp.where(top_mask, jnp.exp(logits - vmax), 0.0)
        gates = e / jnp.sum(e)

        eps = 1e-10
        imp_mean = jnp.sum(gates) / N_EXPERTS
        imp_var = jnp.sum((gates - imp_mean) ** 2) / N_EXPERTS
        load = (gates > 0).astype(jnp.float32)
        load_mean = jnp.sum(load) / N_EXPERTS
        load_var = jnp.sum((load - load_mean) ** 2) / N_EXPERTS
        aux = imp_var / (imp_mean ** 2 + eps) + load_var / (load_mean ** 2 + eps)
        aux_ref[...] = jnp.full((1, 1), aux, jnp.float32)

        mid_ref[1:1 + HALF, :] = ctx_g_ref[...]
        for j in range(NC_ROWS):
            mid_ref[1 + HALF + j:2 + HALF + j, :] = lax.dot_general(
                gates, ctx_c_ref[:, j, :], (((1,), (0,)), ((), ())),
                preferred_element_type=jnp.float32)
        mid_ref[N_CTX:N_CTX + 1, :] = ctx_s

    head = jnp.broadcast_to(mid_ref[0:N_CTX, :][None], (BLK, N_CTX, CTX_DIM))
    rowid = lax.broadcasted_iota(jnp.int32, (BLK, N_CTX, CTX_DIM), 1)
    prow = jnp.broadcast_to(pre_ref[...], (BLK, N_CTX, CTX_DIM))
    out_ref[:, 0:N_CTX, :] = jnp.where(rowid == 0, prow, head)

    ctx_s_b = jnp.broadcast_to(mid_ref[N_CTX:N_CTX + 1, :][None],
                               (BLK, 1, CTX_DIM))
    tail = jnp.concatenate([suf_ref[...], ctx_s_b], axis=1)
    out_ref[:, N_CTX:, :] = pltpu.roll(tail, 1, 1)


def kernel(path, shared, ctx_g, ctx_c, W_shared_w, W_shared_b, w_gate,
           token_prefix, token_suffix, tokenized_prompts):
    ctx_c3 = ctx_c.reshape(N_EXPERTS, NC_ROWS, CTX_DIM)
    b2 = W_shared_b.reshape(1, CTX_DIM)
    prompts, aux = pl.pallas_call(
        _fused_body,
        grid=(N_CLS // BLK,),
        in_specs=[
            pl.BlockSpec((1, 512), lambda c: (0, 0)),
            pl.BlockSpec((1, 256), lambda c: (0, 0)),
            pl.BlockSpec((HALF, CTX_DIM), lambda c: (0, 0)),
            pl.BlockSpec((N_EXPERTS, NC_ROWS, CTX_DIM), lambda c: (0, 0, 0)),
            pl.BlockSpec((CTX_DIM, 256), lambda c: (0, 0)),
            pl.BlockSpec((1, CTX_DIM), lambda c: (0, 0)),
            pl.BlockSpec((512, N_EXPERTS), lambda c: (0, 0)),
            pl.BlockSpec((BLK, 1, CTX_DIM), lambda c: (c, 0, 0)),
            pl.BlockSpec((BLK, SUF, CTX_DIM), lambda c: (c, 0, 0)),
        ],
        out_specs=[
            pl.BlockSpec((BLK, CONTEXT_LEN, CTX_DIM), lambda c: (c, 0, 0)),
            pl.BlockSpec((1, 1), lambda c: (0, 0)),
        ],
        out_shape=[
            jax.ShapeDtypeStruct((N_CLS, CONTEXT_LEN, CTX_DIM), jnp.float32),
            jax.ShapeDtypeStruct((1, 1), jnp.float32),
        ],
        scratch_shapes=[pltpu.VMEM((N_CTX + 8, CTX_DIM), jnp.float32)],
    )(path, shared, ctx_g, ctx_c3, W_shared_w, b2, w_gate,
      token_prefix, token_suffix)
    return prompts, tokenized_prompts, aux.reshape(())


# speedup vs baseline: 4.6157x; 1.0963x over previous
import jax
import jax.numpy as jnp
from jax import lax
from jax.experimental import pallas as pl
from jax.experimental.pallas import tpu as pltpu

N_CLS = 50
N_CTX = 16
CTX_DIM = 768
N_EXPERTS = 64
TOP_K = 4
CONTEXT_LEN = 128
HALF = N_CTX // 2
NC_ROWS = HALF - 1
SUF = CONTEXT_LEN - 1 - N_CTX
BLK = 25


def _fused_body(path_ref, shared_ref, ctx_g_ref, ctx_c_ref, w_ref, b_ref,
                wg_ref, pre_ref, suf_ref, out_ref, aux_ref, mid_ref):
    c = pl.program_id(0)

    @pl.when(c == 0)
    def _compute():
        ctx_s = lax.dot_general(
            shared_ref[...], w_ref[...], (((1,), (1,)), ((), ())),
            preferred_element_type=jnp.float32) + b_ref[...]

        logits = lax.dot_general(
            path_ref[...], wg_ref[...], (((1,), (0,)), ((), ())),
            preferred_element_type=jnp.float32)

        iota = lax.broadcasted_iota(jnp.int32, (1, N_EXPERTS), 1)
        work = logits
        top_mask = jnp.zeros((1, N_EXPERTS), jnp.bool_)
        vmax = jnp.max(work)
        for _ in range(TOP_K):
            m = jnp.max(work)
            sel = jnp.min(jnp.where(work == m, iota, N_EXPERTS))
            mk = iota == sel
            top_mask = jnp.logical_or(top_mask, mk)
            work = jnp.where(mk, -jnp.inf, work)

        e = jnp.where(top_mask, jnp.exp(logits - vmax), 0.0)
        gates = e / jnp.sum(e)

        eps = 1e-10
        imp_mean = jnp.sum(gates) / N_EXPERTS
        imp_var = jnp.sum((gates - imp_mean) ** 2) / N_EXPERTS
        load = (gates > 0).astype(jnp.float32)
        load_mean = jnp.sum(load) / N_EXPERTS
        load_var = jnp.sum((load - load_mean) ** 2) / N_EXPERTS
        aux = imp_var / (imp_mean ** 2 + eps) + load_var / (load_mean ** 2 + eps)
        aux_ref[...] = jnp.full((1, 1), aux, jnp.float32)

        mid_ref[1:1 + HALF, :] = ctx_g_ref[...]
        for j in range(NC_ROWS):
            mid_ref[1 + HALF + j:2 + HALF + j, :] = lax.dot_general(
                gates, ctx_c_ref[:, j, :], (((1,), (0,)), ((), ())),
                preferred_element_type=jnp.float32)
        mid_ref[N_CTX:N_CTX + 1, :] = ctx_s

    head = jnp.broadcast_to(mid_ref[0:N_CTX, :][None], (BLK, N_CTX, CTX_DIM))
    rowid = lax.broadcasted_iota(jnp.int32, (BLK, N_CTX, CTX_DIM), 1)
    prow = jnp.broadcast_to(pre_ref[...], (BLK, N_CTX, CTX_DIM))
    out_ref[:, 0:N_CTX, :] = jnp.where(rowid == 0, prow, head)

    ctx_s_b = jnp.broadcast_to(mid_ref[N_CTX:N_CTX + 1, :][None],
                               (BLK, 1, CTX_DIM))
    tail = jnp.concatenate([suf_ref[...], ctx_s_b], axis=1)
    out_ref[:, N_CTX:, :] = pltpu.roll(tail, 1, 1)


def kernel(path, shared, ctx_g, ctx_c, W_shared_w, W_shared_b, w_gate,
           token_prefix, token_suffix, tokenized_prompts):
    ctx_c3 = ctx_c.reshape(N_EXPERTS, NC_ROWS, CTX_DIM)
    b2 = W_shared_b.reshape(1, CTX_DIM)
    prompts, aux = pl.pallas_call(
        _fused_body,
        grid=(N_CLS // BLK,),
        in_specs=[
            pl.BlockSpec((1, 512), lambda c: (0, 0)),
            pl.BlockSpec((1, 256), lambda c: (0, 0)),
            pl.BlockSpec((HALF, CTX_DIM), lambda c: (0, 0)),
            pl.BlockSpec((N_EXPERTS, NC_ROWS, CTX_DIM), lambda c: (0, 0, 0)),
            pl.BlockSpec((CTX_DIM, 256), lambda c: (0, 0)),
            pl.BlockSpec((1, CTX_DIM), lambda c: (0, 0)),
            pl.BlockSpec((512, N_EXPERTS), lambda c: (0, 0)),
            pl.BlockSpec((BLK, 1, CTX_DIM), lambda c: (c, 0, 0)),
            pl.BlockSpec((BLK, SUF, CTX_DIM), lambda c: (c, 0, 0)),
        ],
        out_specs=[
            pl.BlockSpec((BLK, CONTEXT_LEN, CTX_DIM), lambda c: (c, 0, 0)),
            pl.BlockSpec((1, 1), lambda c: (0, 0)),
        ],
        out_shape=[
            jax.ShapeDtypeStruct((N_CLS, CONTEXT_LEN, CTX_DIM), jnp.float32),
            jax.ShapeDtypeStruct((1, 1), jnp.float32),
        ],
        scratch_shapes=[pltpu.VMEM((N_CTX + 8, CTX_DIM), jnp.float32)],
    )(path, shared, ctx_g, ctx_c3, W_shared_w, b2, w_gate,
      token_prefix, token_suffix)
    return prompts, tokenized_prompts, aux.reshape(())
